# baseline (speedup 1.0000x reference)
import functools

import jax
import jax.numpy as jnp
from jax import lax
from jax.experimental import pallas as pl
from jax.experimental.pallas import tpu as pltpu
from jax.experimental.pallas import tpu_sc as plsc

NUM_OPS = 1000
NUM_BW = 32
EMB = 128
BATCH = 16384


def _tables_body(opt_ref, pt_ref, w1_ref, b1_ref, top_ref, tcat_ref):
    w1 = w1_ref[...]
    top_ref[...] = jnp.dot(opt_ref[...], w1[0:128],
                           preferred_element_type=jnp.float32) + b1_ref[...]
    pt = pt_ref[...]
    tcat_ref[0:32] = jnp.dot(pt, w1[128:256],
                             preferred_element_type=jnp.float32)
    tcat_ref[32:64] = jnp.dot(pt, w1[256:384],
                              preferred_element_type=jnp.float32)
    tcat_ref[64:96] = jnp.dot(pt, w1[384:512],
                              preferred_element_type=jnp.float32)


def _transform_tables(op_table, param_table, W1, b1):
    return pl.pallas_call(
        _tables_body,
        out_shape=(
            jax.ShapeDtypeStruct((NUM_OPS, EMB), jnp.float32),
            jax.ShapeDtypeStruct((3 * NUM_BW, EMB), jnp.float32),
        ),
    )(op_table, param_table, W1, b1.reshape(1, EMB))


def _sc_gather(op2d, top):
    info = plsc.get_sparse_core_info()
    nc, ns = info.num_cores, info.num_subcores
    nw = nc * ns
    ch = 128
    nchunks = BATCH // ch
    steps = nchunks // nw

    mesh = plsc.VectorSubcoreMesh(core_axis_name="c", subcore_axis_name="s")

    @functools.partial(
        pl.kernel,
        mesh=mesh,
        out_type=jax.ShapeDtypeStruct((BATCH, EMB), jnp.float32),
        scratch_types=[
            pltpu.VMEM((steps, ch), jnp.int32),
            pltpu.VMEM((ch, EMB), jnp.float32),
            pltpu.VMEM((ch, EMB), jnp.float32),
            pltpu.VMEM((ch, EMB), jnp.float32),
            pltpu.VMEM((ch, EMB), jnp.float32),
            pltpu.SemaphoreType.DMA,
            pltpu.SemaphoreType.DMA,
        ],
    )
    def k(op_hbm, top_hbm, out_hbm, idx, b0, b1_, b2_, b3_, gsem, ssem):
        wid = lax.axis_index("s") * nc + lax.axis_index("c")
        chunk0 = wid * steps
        pltpu.sync_copy(op_hbm.at[pl.ds(chunk0, steps)], idx)
        bufs = (b0, b1_, b2_, b3_)
        gathers = [None] * steps
        stores = [None] * steps
        for s in range(steps):
            gathers[s] = pltpu.async_copy(top_hbm.at[idx.at[s]], bufs[s], gsem)
        for s in range(steps):
            gathers[s].wait()
            base = (chunk0 + s) * ch
            stores[s] = pltpu.async_copy(
                bufs[s], out_hbm.at[pl.ds(base, ch)], ssem)
        for s in range(steps):
            stores[s].wait()

    return k(op2d, top)


def _mlp2_body(h_ref, p_ref, tcat_ref, w2_ref, b2_ref, o_ref):
    blk = h_ref.shape[0]
    p = p_ref[0]
    io = lax.broadcasted_iota(jnp.int32, (3 * NUM_BW, blk), 0)
    hit = ((io == p[0:1, :]) | (io == p[1:2, :]) | (io == p[2:3, :]))
    oh_t = hit.astype(jnp.float32)
    hp = lax.dot_general(oh_t, tcat_ref[...],
                         (((0,), (0,)), ((), ())),
                         preferred_element_type=jnp.float32)
    h = jnp.maximum(h_ref[...] + hp, 0.0)
    o_ref[...] = jnp.dot(h, w2_ref[...],
                         preferred_element_type=jnp.float32) + b2_ref[...]


def _mlp2(h_op, params3d, tcat, W2, b2):
    blk = 1024
    nblk = BATCH // blk
    return pl.pallas_call(
        _mlp2_body,
        grid=(nblk,),
        in_specs=[
            pl.BlockSpec((blk, EMB), lambda i: (i, 0)),
            pl.BlockSpec((1, 3, blk), lambda i: (i, 0, 0)),
            pl.BlockSpec((3 * NUM_BW, EMB), lambda i: (0, 0)),
            pl.BlockSpec((EMB, EMB), lambda i: (0, 0)),
            pl.BlockSpec((1, EMB), lambda i: (0, 0)),
        ],
        out_specs=pl.BlockSpec((blk, EMB), lambda i: (i, 0)),
        out_shape=jax.ShapeDtypeStruct((BATCH, EMB), jnp.float32),
    )(h_op, params3d, tcat, W2, b2.reshape(1, EMB))


def kernel(op, params, op_table, param_table, W1, b1, W2, b2):
    blk = 1024
    op2d = op.astype(jnp.int32).reshape(BATCH // 128, 128)
    shift = jnp.array([0, NUM_BW, 2 * NUM_BW], dtype=jnp.int32)[:, None]
    params3d = (params.astype(jnp.int32) + shift).reshape(
        3, BATCH // blk, blk).transpose(1, 0, 2)
    top, tcat = _transform_tables(op_table, param_table, W1, b1)
    h_op = _sc_gather(op2d, top)
    return _mlp2(h_op, params3d, tcat, W2, b2)


# speedup vs baseline: 1.1569x; 1.1569x over previous
import functools

import jax
import jax.numpy as jnp
from jax import lax
from jax.experimental import pallas as pl
from jax.experimental.pallas import tpu as pltpu
from jax.experimental.pallas import tpu_sc as plsc

NUM_OPS = 1000
NUM_BW = 32
EMB = 128
BATCH = 16384


def _tables_body(opt_ref, pt_ref, w1_ref, b1_ref, top_ref, tcat_ref):
    w1 = w1_ref[...]
    top_ref[...] = jnp.dot(opt_ref[...], w1[0:128],
                           preferred_element_type=jnp.float32) + b1_ref[...]
    pt = pt_ref[...]
    tcat_ref[0:32] = jnp.dot(pt, w1[128:256],
                             preferred_element_type=jnp.float32)
    tcat_ref[32:64] = jnp.dot(pt, w1[256:384],
                              preferred_element_type=jnp.float32)
    tcat_ref[64:96] = jnp.dot(pt, w1[384:512],
                              preferred_element_type=jnp.float32)


def _transform_tables(op_table, param_table, W1, b1):
    return pl.pallas_call(
        _tables_body,
        out_shape=(
            jax.ShapeDtypeStruct((NUM_OPS, EMB), jnp.float32),
            jax.ShapeDtypeStruct((3 * NUM_BW, EMB), jnp.float32),
        ),
    )(op_table, param_table, W1, b1.reshape(1, EMB))


def _sc_gather(op2d, top):
    info = plsc.get_sparse_core_info()
    nc, ns = info.num_cores, info.num_subcores
    nw = nc * ns
    ch = 128
    nchunks = BATCH // ch
    steps = nchunks // nw

    mesh = plsc.VectorSubcoreMesh(core_axis_name="c", subcore_axis_name="s")

    @functools.partial(
        pl.kernel,
        mesh=mesh,
        out_type=jax.ShapeDtypeStruct((BATCH, EMB), jnp.float32),
        scratch_types=[
            pltpu.VMEM((steps, ch), jnp.int32),
            pltpu.VMEM((ch, EMB), jnp.float32),
            pltpu.VMEM((ch, EMB), jnp.float32),
            pltpu.VMEM((ch, EMB), jnp.float32),
            pltpu.VMEM((ch, EMB), jnp.float32),
            pltpu.SemaphoreType.DMA,
            pltpu.SemaphoreType.DMA,
        ],
    )
    def k(op_hbm, top_hbm, out_hbm, idx, b0, b1_, b2_, b3_, gsem, ssem):
        wid = lax.axis_index("s") * nc + lax.axis_index("c")
        chunk0 = wid * steps
        pltpu.sync_copy(op_hbm.at[pl.ds(chunk0, steps)], idx)
        bufs = (b0, b1_, b2_, b3_)
        gathers = [None] * steps
        stores = [None] * steps
        for s in range(steps):
            gathers[s] = pltpu.async_copy(top_hbm.at[idx.at[s]], bufs[s], gsem)
        for s in range(steps):
            gathers[s].wait()
            base = (chunk0 + s) * ch
            stores[s] = pltpu.async_copy(
                bufs[s], out_hbm.at[pl.ds(base, ch)], ssem)
        for s in range(steps):
            stores[s].wait()

    return k(op2d, top)


def _mlp2_body(h_ref, p_ref, tcat_ref, w2_ref, b2_ref, o_ref):
    blk = h_ref.shape[0]
    p = p_ref[0]
    io = lax.broadcasted_iota(jnp.int32, (3 * NUM_BW, blk), 0)
    hit = ((io == p[0:1, :]) | (io == p[1:2, :]) | (io == p[2:3, :]))
    oh_t = hit.astype(jnp.float32)
    hp = lax.dot_general(oh_t, tcat_ref[...],
                         (((0,), (0,)), ((), ())),
                         preferred_element_type=jnp.float32)
    h = jnp.maximum(h_ref[...] + hp, 0.0)
    o_ref[...] = jnp.dot(h, w2_ref[...],
                         preferred_element_type=jnp.float32) + b2_ref[...]


def _mlp2(h_op, params3d, tcat, W2, b2):
    blk = 4096
    nblk = BATCH // blk
    return pl.pallas_call(
        _mlp2_body,
        grid=(nblk,),
        in_specs=[
            pl.BlockSpec((blk, EMB), lambda i: (i, 0)),
            pl.BlockSpec((1, 3, blk), lambda i: (i, 0, 0)),
            pl.BlockSpec((3 * NUM_BW, EMB), lambda i: (0, 0)),
            pl.BlockSpec((EMB, EMB), lambda i: (0, 0)),
            pl.BlockSpec((1, EMB), lambda i: (0, 0)),
        ],
        out_specs=pl.BlockSpec((blk, EMB), lambda i: (i, 0)),
        out_shape=jax.ShapeDtypeStruct((BATCH, EMB), jnp.float32),
    )(h_op, params3d, tcat, W2, b2.reshape(1, EMB))


def kernel(op, params, op_table, param_table, W1, b1, W2, b2):
    blk = 4096
    op2d = op.astype(jnp.int32).reshape(BATCH // 128, 128)
    shift = jnp.array([0, NUM_BW, 2 * NUM_BW], dtype=jnp.int32)[:, None]
    params3d = (params.astype(jnp.int32) + shift).reshape(
        3, BATCH // blk, blk).transpose(1, 0, 2)
    top, tcat = _transform_tables(op_table, param_table, W1, b1)
    h_op = _sc_gather(op2d, top)
    return _mlp2(h_op, params3d, tcat, W2, b2)


# speedup vs baseline: 1.2065x; 1.0428x over previous
import functools

import jax
import jax.numpy as jnp
from jax import lax
from jax.experimental import pallas as pl
from jax.experimental.pallas import tpu as pltpu
from jax.experimental.pallas import tpu_sc as plsc

NUM_OPS = 1000
NUM_BW = 32
EMB = 128
BATCH = 16384


def _tables_body(opt_ref, pt_ref, w1_ref, b1_ref, top_ref, tcat_ref):
    w1 = w1_ref[...]
    top_ref[...] = jnp.dot(opt_ref[...], w1[0:128],
                           preferred_element_type=jnp.float32) + b1_ref[...]
    pt = pt_ref[...]
    tcat_ref[0:32] = jnp.dot(pt, w1[128:256],
                             preferred_element_type=jnp.float32)
    tcat_ref[32:64] = jnp.dot(pt, w1[256:384],
                              preferred_element_type=jnp.float32)
    tcat_ref[64:96] = jnp.dot(pt, w1[384:512],
                              preferred_element_type=jnp.float32)


def _transform_tables(op_table, param_table, W1, b1):
    return pl.pallas_call(
        _tables_body,
        out_shape=(
            jax.ShapeDtypeStruct((NUM_OPS, EMB), jnp.float32),
            jax.ShapeDtypeStruct((3 * NUM_BW, EMB), jnp.float32),
        ),
    )(op_table, param_table, W1, b1.reshape(1, EMB))


def _sc_gather(op2d, top):
    info = plsc.get_sparse_core_info()
    nc, ns = info.num_cores, info.num_subcores
    nw = nc * ns
    ch = 128
    nchunks = BATCH // ch
    steps = nchunks // nw

    mesh = plsc.VectorSubcoreMesh(core_axis_name="c", subcore_axis_name="s")

    @functools.partial(
        pl.kernel,
        mesh=mesh,
        out_type=jax.ShapeDtypeStruct((BATCH, EMB), jnp.float32),
        scratch_types=[
            pltpu.VMEM((steps, ch), jnp.int32),
            pltpu.VMEM((ch, EMB), jnp.float32),
            pltpu.VMEM((ch, EMB), jnp.float32),
            pltpu.VMEM((ch, EMB), jnp.float32),
            pltpu.VMEM((ch, EMB), jnp.float32),
            pltpu.SemaphoreType.DMA,
            pltpu.SemaphoreType.DMA,
        ],
    )
    def k(op_hbm, top_hbm, out_hbm, idx, b0, b1_, b2_, b3_, gsem, ssem):
        wid = lax.axis_index("s") * nc + lax.axis_index("c")
        chunk0 = wid * steps
        pltpu.sync_copy(op_hbm.at[pl.ds(chunk0, steps)], idx)
        bufs = (b0, b1_, b2_, b3_)
        gathers = [None] * steps
        stores = [None] * steps
        for s in range(steps):
            gathers[s] = pltpu.async_copy(top_hbm.at[idx.at[s]], bufs[s], gsem)
        for s in range(steps):
            gathers[s].wait()
            base = (chunk0 + s) * ch
            stores[s] = pltpu.async_copy(
                bufs[s], out_hbm.at[pl.ds(base, ch)], ssem)
        for s in range(steps):
            stores[s].wait()

    return k(op2d, top)


def _mlp2_body(h_ref, p_ref, tcat_ref, w2_ref, b2_ref, o_ref):
    blk = h_ref.shape[0]
    p = p_ref[0]
    io = lax.broadcasted_iota(jnp.int32, (3 * NUM_BW, blk), 0)
    hit = ((io == p[0:1, :]) | (io == p[1:2, :]) | (io == p[2:3, :]))
    oh_t = hit.astype(jnp.float32)
    hp = lax.dot_general(oh_t, tcat_ref[...],
                         (((0,), (0,)), ((), ())),
                         preferred_element_type=jnp.float32)
    h = jnp.maximum(h_ref[...] + hp, 0.0)
    o_ref[...] = jnp.dot(h, w2_ref[...],
                         preferred_element_type=jnp.float32) + b2_ref[...]


def _mlp2(h_op, params3d, tcat, W2, b2):
    blk = 8192
    nblk = BATCH // blk
    return pl.pallas_call(
        _mlp2_body,
        grid=(nblk,),
        in_specs=[
            pl.BlockSpec((blk, EMB), lambda i: (i, 0)),
            pl.BlockSpec((1, 3, blk), lambda i: (i, 0, 0)),
            pl.BlockSpec((3 * NUM_BW, EMB), lambda i: (0, 0)),
            pl.BlockSpec((EMB, EMB), lambda i: (0, 0)),
            pl.BlockSpec((1, EMB), lambda i: (0, 0)),
        ],
        out_specs=pl.BlockSpec((blk, EMB), lambda i: (i, 0)),
        out_shape=jax.ShapeDtypeStruct((BATCH, EMB), jnp.float32),
    )(h_op, params3d, tcat, W2, b2.reshape(1, EMB))


def kernel(op, params, op_table, param_table, W1, b1, W2, b2):
    blk = 8192
    op2d = op.astype(jnp.int32).reshape(BATCH // 128, 128)
    shift = jnp.array([0, NUM_BW, 2 * NUM_BW], dtype=jnp.int32)[:, None]
    params3d = (params.astype(jnp.int32) + shift).reshape(
        3, BATCH // blk, blk).transpose(1, 0, 2)
    top, tcat = _transform_tables(op_table, param_table, W1, b1)
    h_op = _sc_gather(op2d, top)
    return _mlp2(h_op, params3d, tcat, W2, b2)
